# baseline (speedup 1.0000x reference)
import functools

import jax
import jax.numpy as jnp
import numpy as np
from jax import lax
from jax.experimental import pallas as pl
from jax.experimental.pallas import tpu as pltpu
from jax.experimental.pallas import tpu_sc as plsc

N_PTS0 = 32
N_PTS1 = 8
DIM_LOI = 128
DIM_FC = 1024
HH = 128
WW = 128
HW = HH * WW
NBUF = 2
B = 2
NL = 5000
NLINES = B * NL
NPAD = 10240
LPW = NPAD // 32

_LAM = np.linspace(0.0, 1.0, N_PTS0).astype(np.float32)


def _fc1_body(f_ref, w_ref, b_ref, o_ref):
    f = f_ref[0]
    xb = lax.dot_general(
        f, w_ref[...], (((0,), (1,)), ((), ())),
        preferred_element_type=jnp.float32) + b_ref[...]
    sh = jnp.concatenate([xb[1:], xb[-1:]], axis=0)
    rid = lax.broadcasted_iota(jnp.int32, (xb.shape[0], 1), 0)
    sh = jnp.where(rid % WW == WW - 1, xb, sh)
    o_ref[...] = jnp.concatenate([xb, sh], axis=1)


def _fc1(feature_r, W1, b1r):
    HWT = 2048
    ntile = HW // HWT
    return pl.pallas_call(
        _fc1_body,
        grid=(B * ntile,),
        in_specs=[
            pl.BlockSpec((1, 256, HWT), lambda i: (i // 8, 0, i % 8)),
            pl.BlockSpec((DIM_LOI, 256), lambda i: (0, 0)),
            pl.BlockSpec((1, DIM_LOI), lambda i: (0, 0)),
        ],
        out_specs=pl.BlockSpec((HWT, 2 * DIM_LOI), lambda i: (i, 0)),
        out_shape=jax.ShapeDtypeStruct((B * HW, 2 * DIM_LOI), jnp.float32),
    )(feature_r, W1, b1r)


def _floor_f32(x):
    t = x.astype(jnp.int32).astype(jnp.float32)
    return jnp.where(t > x, t - 1.0, t)


_BCAST_DN = lax.GatherDimensionNumbers(
    offset_dims=(), collapsed_slice_dims=(0,), start_index_map=(0,))


def _bcast(v, lane):
    idx = jnp.full((16, 1), lane, jnp.int32)
    return lax.gather(v, idx, _BCAST_DN, (1,),
                      mode=lax.GatherScatterMode.PROMISE_IN_BOUNDS)


def _sc_gather_pool(table, lx0, ly0, lx1, ly1):
    nc, ns = 2, 16
    mesh = plsc.VectorSubcoreMesh(core_axis_name="c", subcore_axis_name="s",
                                  num_cores=nc, num_subcores=ns)

    @functools.partial(
        pl.kernel,
        out_type=jax.ShapeDtypeStruct((NPAD, DIM_LOI * N_PTS1), jnp.float32),
        mesh=mesh,
        scratch_types=[
            [pltpu.VMEM((LPW,), jnp.float32)] * 4,
            [pltpu.VMEM((64,), jnp.int32)] * NBUF,
            [pltpu.VMEM((128,), jnp.float32)] * NBUF,
            [pltpu.VMEM((64, 2 * DIM_LOI), jnp.float32)] * NBUF,
            [pltpu.VMEM((DIM_LOI * N_PTS1,), jnp.float32)] * NBUF,
            [pltpu.SemaphoreType.DMA] * NBUF,
            [pltpu.SemaphoreType.DMA] * NBUF,
        ],
    )
    def k(table_hbm, lx0_hbm, ly0_hbm, lx1_hbm, ly1_hbm, feat_hbm,
          lines_v, idx_v, w_v, rows_v, out_v, gsem, osem):
        wid = lax.axis_index("s") * nc + lax.axis_index("c")
        base = wid * LPW
        for src, dst in zip((lx0_hbm, ly0_hbm, lx1_hbm, ly1_hbm), lines_v):
            pltpu.sync_copy(src.at[pl.ds(base, LPW)], dst)

        iotaf = lax.iota(jnp.int32, 16).astype(jnp.float32)

        def idx_calc(i, slot):
            gl = base + i
            g16 = (i // 16) * 16
            lane = i % 16
            x0 = _bcast(lines_v[0][pl.ds(g16, 16)], lane)
            y0 = _bcast(lines_v[1][pl.ds(g16, 16)], lane)
            x1 = _bcast(lines_v[2][pl.ds(g16, 16)], lane)
            y1 = _bcast(lines_v[3][pl.ds(g16, 16)], lane)
            boff = jnp.where(jnp.full((16,), gl, jnp.int32) >= NL,
                             HW, 0).astype(jnp.int32)
            for pg in range(2):
                lam = (iotaf + (16.0 * pg)) * np.float32(1.0 / 31.0)
                om = 1.0 - lam
                px = x0 * lam + x1 * om - 0.5
                py = y0 * lam + y1 * om - 0.5
                px0 = jnp.clip(_floor_f32(px), 0.0, HH - 1.0)
                py0 = jnp.clip(_floor_f32(py), 0.0, WW - 1.0)
                px1 = jnp.minimum(px0 + 1.0, HH - 1.0)
                py1 = jnp.minimum(py0 + 1.0, WW - 1.0)
                ix0 = px0.astype(jnp.int32)
                iy0 = py0.astype(jnp.int32)
                ix1 = px1.astype(jnp.int32)
                iy1 = py1.astype(jnp.int32)
                wx0 = px - px0
                wx1 = px1 - px
                wy0 = py - py0
                wy1 = py1 - py
                o = pg * 16
                idx_v[slot][pl.ds(o, 16)] = boff + ix0 * WW + iy0
                idx_v[slot][pl.ds(32 + o, 16)] = boff + ix1 * WW + iy0
                w_v[slot][pl.ds(o, 16)] = wx1 * wy1
                w_v[slot][pl.ds(32 + o, 16)] = wx1 * wy0
                w_v[slot][pl.ds(64 + o, 16)] = wx0 * wy1
                w_v[slot][pl.ds(96 + o, 16)] = wx0 * wy0

        def start_gather(slot):
            pltpu.async_copy(table_hbm.at[idx_v[slot]], rows_v[slot],
                             gsem[slot])

        def compute_line(i, slot):
            gl = base + i
            rows = rows_v[slot]
            for pg in range(2):
                wch = [w_v[slot][pl.ds(kk * 32 + pg * 16, 16)]
                       for kk in range(4)]

                @plsc.parallel_loop(0, 4, 1, unroll=2)
                def group_body(j2, pg=pg, wch=wch):
                    j = pg * 4 + j2
                    wv = [[_bcast(wch[kk], 4 * j2 + q) for kk in range(4)]
                          for q in range(4)]
                    for c in range(8):
                        m = None
                        for q in range(4):
                            p = 16 * pg + 4 * j2 + q
                            acc = ((wv[q][0] * rows[p, pl.ds(c * 16, 16)]
                                    + wv[q][1] * rows[p,
                                                      pl.ds(128 + c * 16, 16)])
                                   + (wv[q][2] * rows[32 + p,
                                                      pl.ds(c * 16, 16)]
                                      + wv[q][3] * rows[32 + p,
                                                        pl.ds(128 + c * 16,
                                                              16)]))
                            m = acc if q == 0 else jnp.maximum(m, acc)
                        out_v[slot][pl.ds(j * DIM_LOI + c * 16, 16)] = m
            pltpu.async_copy(out_v[slot], feat_hbm.at[gl], osem[slot])

        for b in range(NBUF):
            idx_calc(b, b)
            start_gather(b)

        def pair_body(g, carry):
            for b in range(NBUF):
                i = NBUF * g + b
                pltpu.make_async_copy(table_hbm.at[idx_v[b]], rows_v[b],
                                      gsem[b]).wait()

                @pl.when(g > 0)
                def _():
                    pltpu.make_async_copy(out_v[b], feat_hbm.at[base],
                                          osem[b]).wait()

                compute_line(i, b)
                nxt = jnp.minimum(i + NBUF, LPW - 1)
                idx_calc(nxt, b)
                start_gather(b)
            return carry

        lax.fori_loop(0, LPW // NBUF, pair_body, 0)
        for b in range(NBUF):
            pltpu.make_async_copy(table_hbm.at[idx_v[b]], rows_v[b],
                                  gsem[b]).wait()
            pltpu.make_async_copy(out_v[b], feat_hbm.at[base],
                                  osem[b]).wait()

    return k(table, lx0, ly0, lx1, ly1)


def _mlp_body(f_ref, wa_ref, ba_ref, wb_ref, bb_ref, wc_ref, bc_ref,
              lo_ref, bf_ref):
    h1 = jax.nn.relu(lax.dot_general(
        f_ref[...], wa_ref[...], (((1,), (1,)), ((), ())),
        preferred_element_type=jnp.float32) + ba_ref[...])
    h2 = jax.nn.relu(lax.dot_general(
        h1, wb_ref[...], (((1,), (1,)), ((), ())),
        preferred_element_type=jnp.float32) + bb_ref[...])
    lg = lax.dot_general(
        h2, wc_ref[...], (((1,), (1,)), ((), ())),
        preferred_element_type=jnp.float32) + bc_ref[...]
    l4 = lg[:, :4]
    lo_ref[...] = l4
    mx = jnp.max(l4, axis=1, keepdims=True)
    e = jnp.exp(l4 - mx)
    s = e / jnp.sum(e, axis=1, keepdims=True)
    cond = ((s[:, 1:2] > 0.25) | (s[:, 2:3] > 0.25)
            | (s[:, 3:4] > 0.25)) & (s[:, 0:1] < 0.25)
    bf_ref[...] = jnp.where(cond, 1.0, 0.0)


def _mlp(feat, W2ap, b2ar, W2b, b2br, W2cp, b2cp):
    RT = 1024
    return pl.pallas_call(
        _mlp_body,
        grid=(NPAD // RT,),
        in_specs=[
            pl.BlockSpec((RT, DIM_FC), lambda i: (i, 0)),
            pl.BlockSpec((DIM_FC, DIM_FC), lambda i: (0, 0)),
            pl.BlockSpec((1, DIM_FC), lambda i: (0, 0)),
            pl.BlockSpec((DIM_FC, DIM_FC), lambda i: (0, 0)),
            pl.BlockSpec((1, DIM_FC), lambda i: (0, 0)),
            pl.BlockSpec((128, DIM_FC), lambda i: (0, 0)),
            pl.BlockSpec((1, 128), lambda i: (0, 0)),
        ],
        out_specs=[
            pl.BlockSpec((RT, 4), lambda i: (i, 0)),
            pl.BlockSpec((RT, 1), lambda i: (i, 0)),
        ],
        out_shape=[
            jax.ShapeDtypeStruct((NPAD, 4), jnp.float32),
            jax.ShapeDtypeStruct((NPAD, 1), jnp.float32),
        ],
    )(feat, W2ap, b2ar, W2b, b2br, W2cp, b2cp)


def _nms_body(j_ref, o_ref):
    a = j_ref[0]
    m = jnp.max(a, axis=0)
    neg = jnp.full((2, WW), -jnp.inf, jnp.float32)
    mp = jnp.concatenate([neg, m, neg], axis=0)
    rm = mp[0:HH]
    for d in range(1, 5):
        rm = jnp.maximum(rm, mp[d:d + HH])
    negc = jnp.full((HH, 2), -jnp.inf, jnp.float32)
    cp = jnp.concatenate([negc, rm, negc], axis=1)
    cm = cp[:, 0:WW]
    for d in range(1, 5):
        cm = jnp.maximum(cm, cp[:, d:d + WW])
    keep = (a == cm[None, :, :]).astype(jnp.float32)
    o_ref[0] = a * keep


def _nms(jmap):
    NT = jmap.shape[1]
    return pl.pallas_call(
        _nms_body,
        grid=(B,),
        in_specs=[pl.BlockSpec((1, NT, HH, WW), lambda i: (i, 0, 0, 0))],
        out_specs=pl.BlockSpec((1, NT, HH, WW), lambda i: (i, 0, 0, 0)),
        out_shape=jax.ShapeDtypeStruct(jmap.shape, jnp.float32),
    )(jmap)


def kernel(feature, jmap, lines, W1, b1, W2a, b2a, W2b, b2b, W2c, b2c):
    feature_r = feature.reshape(B, 256, HW)
    table = _fc1(feature_r, W1, b1.reshape(1, DIM_LOI))

    lf = lines.reshape(NLINES, 4)
    lf = jnp.pad(lf, ((0, NPAD - NLINES), (0, 0)))

    feat = _sc_gather_pool(table, lf[:, 0], lf[:, 1], lf[:, 2], lf[:, 3])

    W2ap = W2a.reshape(DIM_FC, DIM_LOI, N_PTS1).transpose(0, 2, 1).reshape(
        DIM_FC, DIM_FC)
    W2cp = jnp.pad(W2c, ((0, 124), (0, 0)))
    b2cp = jnp.pad(b2c.reshape(1, 4), ((0, 0), (0, 124)))
    logits_p, bf = _mlp(feat, W2ap, b2a.reshape(1, DIM_FC), W2b,
                        b2b.reshape(1, DIM_FC), W2cp, b2cp)

    logits = logits_p[:NLINES]
    b = bf[:NLINES, 0] > 0.5

    jmap_nms = _nms(jmap).reshape(B, jmap.shape[1], HW)
    return logits, jmap_nms, b


# speedup vs baseline: 1.4773x; 1.4773x over previous
import functools

import jax
import jax.numpy as jnp
import numpy as np
from jax import lax
from jax.experimental import pallas as pl
from jax.experimental.pallas import tpu as pltpu
from jax.experimental.pallas import tpu_sc as plsc

N_PTS0 = 32
N_PTS1 = 8
DIM_LOI = 128
DIM_FC = 1024
HH = 128
WW = 128
HW = HH * WW
NBUF = 2
B = 2
NL = 5000
NLINES = B * NL
NPAD = 10240
LPW = NPAD // 32

_LAM = np.linspace(0.0, 1.0, N_PTS0).astype(np.float32)


def _fc1_body(f_ref, w_ref, b_ref, o_ref):
    f = f_ref[0]
    o_ref[...] = lax.dot_general(
        f, w_ref[...], (((0,), (1,)), ((), ())),
        preferred_element_type=jnp.float32) + b_ref[...]


def _fc1(feature_r, W1, b1r):
    HWT = 2048
    ntile = HW // HWT
    return pl.pallas_call(
        _fc1_body,
        grid=(B * ntile,),
        in_specs=[
            pl.BlockSpec((1, 256, HWT), lambda i: (i // 8, 0, i % 8)),
            pl.BlockSpec((DIM_LOI, 256), lambda i: (0, 0)),
            pl.BlockSpec((1, DIM_LOI), lambda i: (0, 0)),
        ],
        out_specs=pl.BlockSpec((HWT, DIM_LOI), lambda i: (i, 0)),
        out_shape=jax.ShapeDtypeStruct((B * HW, DIM_LOI), jnp.float32),
    )(feature_r, W1, b1r)


def _floor_f32(x):
    t = x.astype(jnp.int32).astype(jnp.float32)
    return jnp.where(t > x, t - 1.0, t)


_BCAST_DN = lax.GatherDimensionNumbers(
    offset_dims=(), collapsed_slice_dims=(0,), start_index_map=(0,))


def _bcast(v, lane):
    idx = jnp.full((16, 1), lane, jnp.int32)
    return lax.gather(v, idx, _BCAST_DN, (1,),
                      mode=lax.GatherScatterMode.PROMISE_IN_BOUNDS)


def _sc_gather_pool(table, lx0, ly0, lx1, ly1):
    nc, ns = 2, 16
    mesh = plsc.VectorSubcoreMesh(core_axis_name="c", subcore_axis_name="s",
                                  num_cores=nc, num_subcores=ns)

    @functools.partial(
        pl.kernel,
        out_type=jax.ShapeDtypeStruct((NPAD, DIM_LOI * N_PTS1), jnp.float32),
        mesh=mesh,
        scratch_types=[
            [pltpu.VMEM((LPW,), jnp.float32)] * 4,
            [pltpu.VMEM((128,), jnp.int32)] * NBUF,
            [pltpu.VMEM((128,), jnp.float32)] * NBUF,
            [pltpu.VMEM((128, DIM_LOI), jnp.float32)] * NBUF,
            [pltpu.VMEM((DIM_LOI * N_PTS1,), jnp.float32)] * NBUF,
            [pltpu.SemaphoreType.DMA] * NBUF,
            [pltpu.SemaphoreType.DMA] * NBUF,
        ],
    )
    def k(table_hbm, lx0_hbm, ly0_hbm, lx1_hbm, ly1_hbm, feat_hbm,
          lines_v, idx_v, w_v, rows_v, out_v, gsem, osem):
        wid = lax.axis_index("s") * nc + lax.axis_index("c")
        base = wid * LPW
        for src, dst in zip((lx0_hbm, ly0_hbm, lx1_hbm, ly1_hbm), lines_v):
            pltpu.sync_copy(src.at[pl.ds(base, LPW)], dst)

        iotaf = lax.iota(jnp.int32, 16).astype(jnp.float32)

        def idx_calc(i, slot):
            gl = base + i
            g16 = (i // 16) * 16
            lane = i % 16
            x0 = _bcast(lines_v[0][pl.ds(g16, 16)], lane)
            y0 = _bcast(lines_v[1][pl.ds(g16, 16)], lane)
            x1 = _bcast(lines_v[2][pl.ds(g16, 16)], lane)
            y1 = _bcast(lines_v[3][pl.ds(g16, 16)], lane)
            boff = jnp.where(jnp.full((16,), gl, jnp.int32) >= NL,
                             HW, 0).astype(jnp.int32)
            for pg in range(2):
                lam = (iotaf + (16.0 * pg)) * np.float32(1.0 / 31.0)
                om = 1.0 - lam
                px = x0 * lam + x1 * om - 0.5
                py = y0 * lam + y1 * om - 0.5
                px0 = jnp.clip(_floor_f32(px), 0.0, HH - 1.0)
                py0 = jnp.clip(_floor_f32(py), 0.0, WW - 1.0)
                px1 = jnp.minimum(px0 + 1.0, HH - 1.0)
                py1 = jnp.minimum(py0 + 1.0, WW - 1.0)
                ix0 = px0.astype(jnp.int32)
                iy0 = py0.astype(jnp.int32)
                ix1 = px1.astype(jnp.int32)
                iy1 = py1.astype(jnp.int32)
                wx0 = px - px0
                wx1 = px1 - px
                wy0 = py - py0
                wy1 = py1 - py
                o = pg * 16
                idx_v[slot][pl.ds(o, 16)] = boff + ix0 * WW + iy0
                idx_v[slot][pl.ds(32 + o, 16)] = boff + ix1 * WW + iy0
                idx_v[slot][pl.ds(64 + o, 16)] = boff + ix0 * WW + iy1
                idx_v[slot][pl.ds(96 + o, 16)] = boff + ix1 * WW + iy1
                w_v[slot][pl.ds(o, 16)] = wx1 * wy1
                w_v[slot][pl.ds(32 + o, 16)] = wx0 * wy1
                w_v[slot][pl.ds(64 + o, 16)] = wx1 * wy0
                w_v[slot][pl.ds(96 + o, 16)] = wx0 * wy0

        def start_gather(slot):
            pltpu.async_copy(table_hbm.at[idx_v[slot]], rows_v[slot],
                             gsem[slot])

        def compute_line(i, slot):
            gl = base + i
            rows = rows_v[slot]
            for pg in range(2):
                wch = [w_v[slot][pl.ds(kk * 32 + pg * 16, 16)]
                       for kk in range(4)]

                @plsc.parallel_loop(0, 4, 1, unroll=2)
                def group_body(j2, pg=pg, wch=wch):
                    j = pg * 4 + j2
                    wv = [[_bcast(wch[kk], 4 * j2 + q) for kk in range(4)]
                          for q in range(4)]
                    for c in range(8):
                        m = None
                        for q in range(4):
                            p = 16 * pg + 4 * j2 + q
                            acc = ((wv[q][0] * rows[p, pl.ds(c * 16, 16)]
                                    + wv[q][1] * rows[32 + p,
                                                      pl.ds(c * 16, 16)])
                                   + (wv[q][2] * rows[64 + p,
                                                      pl.ds(c * 16, 16)]
                                      + wv[q][3] * rows[96 + p,
                                                        pl.ds(c * 16, 16)]))
                            m = acc if q == 0 else jnp.maximum(m, acc)
                        out_v[slot][pl.ds(j * DIM_LOI + c * 16, 16)] = m
            pltpu.async_copy(out_v[slot], feat_hbm.at[gl], osem[slot])

        for b in range(NBUF):
            idx_calc(b, b)
            start_gather(b)

        def pair_body(g, carry):
            for b in range(NBUF):
                i = NBUF * g + b
                pltpu.make_async_copy(table_hbm.at[idx_v[b]], rows_v[b],
                                      gsem[b]).wait()

                @pl.when(g > 0)
                def _():
                    pltpu.make_async_copy(out_v[b], feat_hbm.at[base],
                                          osem[b]).wait()

                compute_line(i, b)
                nxt = jnp.minimum(i + NBUF, LPW - 1)
                idx_calc(nxt, b)
                start_gather(b)
            return carry

        lax.fori_loop(0, LPW // NBUF, pair_body, 0)
        for b in range(NBUF):
            pltpu.make_async_copy(table_hbm.at[idx_v[b]], rows_v[b],
                                  gsem[b]).wait()
            pltpu.make_async_copy(out_v[b], feat_hbm.at[base],
                                  osem[b]).wait()

    return k(table, lx0, ly0, lx1, ly1)


def _mlp_body(f_ref, wa_ref, ba_ref, wb_ref, bb_ref, wc_ref, bc_ref,
              lo_ref, bf_ref):
    h1 = jax.nn.relu(lax.dot_general(
        f_ref[...], wa_ref[...], (((1,), (1,)), ((), ())),
        preferred_element_type=jnp.float32) + ba_ref[...])
    h2 = jax.nn.relu(lax.dot_general(
        h1, wb_ref[...], (((1,), (1,)), ((), ())),
        preferred_element_type=jnp.float32) + bb_ref[...])
    lg = lax.dot_general(
        h2, wc_ref[...], (((1,), (1,)), ((), ())),
        preferred_element_type=jnp.float32) + bc_ref[...]
    l4 = lg[:, :4]
    lo_ref[...] = l4
    mx = jnp.max(l4, axis=1, keepdims=True)
    e = jnp.exp(l4 - mx)
    s = e / jnp.sum(e, axis=1, keepdims=True)
    cond = ((s[:, 1:2] > 0.25) | (s[:, 2:3] > 0.25)
            | (s[:, 3:4] > 0.25)) & (s[:, 0:1] < 0.25)
    bf_ref[...] = jnp.where(cond, 1.0, 0.0)


def _mlp(feat, W2ap, b2ar, W2b, b2br, W2cp, b2cp):
    RT = 1024
    return pl.pallas_call(
        _mlp_body,
        grid=(NPAD // RT,),
        in_specs=[
            pl.BlockSpec((RT, DIM_FC), lambda i: (i, 0)),
            pl.BlockSpec((DIM_FC, DIM_FC), lambda i: (0, 0)),
            pl.BlockSpec((1, DIM_FC), lambda i: (0, 0)),
            pl.BlockSpec((DIM_FC, DIM_FC), lambda i: (0, 0)),
            pl.BlockSpec((1, DIM_FC), lambda i: (0, 0)),
            pl.BlockSpec((128, DIM_FC), lambda i: (0, 0)),
            pl.BlockSpec((1, 128), lambda i: (0, 0)),
        ],
        out_specs=[
            pl.BlockSpec((RT, 4), lambda i: (i, 0)),
            pl.BlockSpec((RT, 1), lambda i: (i, 0)),
        ],
        out_shape=[
            jax.ShapeDtypeStruct((NPAD, 4), jnp.float32),
            jax.ShapeDtypeStruct((NPAD, 1), jnp.float32),
        ],
    )(feat, W2ap, b2ar, W2b, b2br, W2cp, b2cp)


def _nms_body(j_ref, o_ref):
    a = j_ref[0]
    m = jnp.max(a, axis=0)
    neg = jnp.full((2, WW), -jnp.inf, jnp.float32)
    mp = jnp.concatenate([neg, m, neg], axis=0)
    rm = mp[0:HH]
    for d in range(1, 5):
        rm = jnp.maximum(rm, mp[d:d + HH])
    negc = jnp.full((HH, 2), -jnp.inf, jnp.float32)
    cp = jnp.concatenate([negc, rm, negc], axis=1)
    cm = cp[:, 0:WW]
    for d in range(1, 5):
        cm = jnp.maximum(cm, cp[:, d:d + WW])
    keep = (a == cm[None, :, :]).astype(jnp.float32)
    o_ref[0] = a * keep


def _nms(jmap):
    NT = jmap.shape[1]
    return pl.pallas_call(
        _nms_body,
        grid=(B,),
        in_specs=[pl.BlockSpec((1, NT, HH, WW), lambda i: (i, 0, 0, 0))],
        out_specs=pl.BlockSpec((1, NT, HH, WW), lambda i: (i, 0, 0, 0)),
        out_shape=jax.ShapeDtypeStruct(jmap.shape, jnp.float32),
    )(jmap)


def kernel(feature, jmap, lines, W1, b1, W2a, b2a, W2b, b2b, W2c, b2c):
    feature_r = feature.reshape(B, 256, HW)
    table = _fc1(feature_r, W1, b1.reshape(1, DIM_LOI))

    lf = lines.reshape(NLINES, 4)
    lf = jnp.pad(lf, ((0, NPAD - NLINES), (0, 0)))

    feat = _sc_gather_pool(table, lf[:, 0], lf[:, 1], lf[:, 2], lf[:, 3])

    W2ap = W2a.reshape(DIM_FC, DIM_LOI, N_PTS1).transpose(0, 2, 1).reshape(
        DIM_FC, DIM_FC)
    W2cp = jnp.pad(W2c, ((0, 124), (0, 0)))
    b2cp = jnp.pad(b2c.reshape(1, 4), ((0, 0), (0, 124)))
    logits_p, bf = _mlp(feat, W2ap, b2a.reshape(1, DIM_FC), W2b,
                        b2b.reshape(1, DIM_FC), W2cp, b2cp)

    logits = logits_p[:NLINES]
    b = bf[:NLINES, 0] > 0.5

    jmap_nms = _nms(jmap).reshape(B, jmap.shape[1], HW)
    return logits, jmap_nms, b
